# baseline (speedup 1.0000x reference)
import functools
import jax
import jax.numpy as jnp
from jax import lax
from jax.experimental import pallas as pl
from jax.experimental.pallas import tpu as pltpu
from jax.experimental.pallas import tpu_sc as plsc

_NC, _NS = 2, 16
_NW = _NC * _NS


def _make_sc_copy(rows, d, dtype):
    per = rows // _NW
    mesh = plsc.VectorSubcoreMesh(
        core_axis_name="c", subcore_axis_name="s",
        num_cores=_NC, num_subcores=_NS,
    )

    @functools.partial(
        pl.kernel,
        out_type=jax.ShapeDtypeStruct((rows, d), dtype),
        mesh=mesh,
        scratch_types=[
            pltpu.VMEM((per, d), dtype),
            pltpu.SemaphoreType.DMA,
        ],
    )
    def sc_copy(pe_hbm, out_hbm, buf, sem):
        wid = lax.axis_index("s") * _NC + lax.axis_index("c")
        base = wid * per
        pltpu.sync_copy(pe_hbm.at[pl.ds(base, per)], buf)
        pltpu.sync_copy(buf, out_hbm.at[pl.ds(base, per)])

    return sc_copy


def kernel(x, pe):
    seq_len = x.shape[1]
    d = pe.shape[2]
    pe2 = pe.reshape(pe.shape[1], d)
    out = _make_sc_copy(seq_len, d, pe.dtype)(pe2)
    return out.reshape(1, seq_len, d)
